# baseline (speedup 1.0000x reference)
import functools

import jax
import jax.numpy as jnp
from jax import lax
from jax.experimental import pallas as pl
from jax.experimental.pallas import tpu as pltpu
from jax.experimental.pallas import tpu_sc as plsc

MINI_BATCH = 16384
STATE_DIM = 64
NC = 2
NS = 16
NW = NC * NS
B_PER_W = MINI_BATCH // NW
CHUNK = 128
NCHUNK = B_PER_W // CHUNK


def _sample_body(state_hbm, next_hbm, rew_hbm, msk_hbm, act_hbm, idx_hbm,
                 out_state, out_act, out_rew, out_next, out_msk,
                 idx_v, idx_f, rew_v, msk_v, act_v, sem,
                 sem2, sem3):
    wid = lax.axis_index("s") * NC + lax.axis_index("c")
    base = wid * B_PER_W

    pltpu.sync_copy(idx_hbm.at[pl.ds(base, B_PER_W)], idx_f)
    for c in range(NCHUNK):
        pltpu.sync_copy(idx_hbm.at[pl.ds(base + c * CHUNK, CHUNK)],
                        idx_v.at[c])

    copies = []
    for c in range(NCHUNK):
        ids = idx_v.at[c]
        copies.append(pltpu.async_copy(rew_hbm.at[ids], rew_v.at[c], sem3))
        copies.append(pltpu.async_copy(msk_hbm.at[ids], msk_v.at[c], sem3))
        copies.append(pltpu.async_copy(act_hbm.at[ids], act_v.at[c], sem3))

    @pl.loop(0, B_PER_W // 16)
    def _grp(g):
        v = idx_f[pl.ds(g * 16, 16)]
        for j in range(16):
            r = v[j]
            pltpu.async_copy(state_hbm.at[r],
                             out_state.at[base + g * 16 + j], sem)
            pltpu.async_copy(next_hbm.at[r],
                             out_next.at[base + g * 16 + j], sem2)

    pltpu.make_async_copy(state_hbm.at[pl.ds(0, B_PER_W)],
                          out_state.at[pl.ds(base, B_PER_W)], sem).wait()
    pltpu.make_async_copy(next_hbm.at[pl.ds(0, B_PER_W)],
                          out_next.at[pl.ds(base, B_PER_W)], sem2).wait()
    for cp in copies:
        cp.wait()

    for c in range(NCHUNK):
        off = base + c * CHUNK
        pltpu.sync_copy(rew_v.at[c], out_rew.at[pl.ds(off, CHUNK)])
        pltpu.sync_copy(msk_v.at[c], out_msk.at[pl.ds(off, CHUNK)])
        pltpu.sync_copy(act_v.at[c], out_act.at[pl.ds(off, CHUNK)])


@jax.jit
def kernel(state, next_state, reward, masks, action, idx):
    idx = idx.astype(jnp.int32)
    act_dtype = action.dtype
    mesh = plsc.VectorSubcoreMesh(core_axis_name="c", subcore_axis_name="s")
    run = pl.kernel(
        _sample_body,
        mesh=mesh,
        compiler_params=pltpu.CompilerParams(use_tc_tiling_on_sc=True),
        out_type=[
            jax.ShapeDtypeStruct((MINI_BATCH, STATE_DIM), jnp.float32),
            jax.ShapeDtypeStruct((MINI_BATCH,), act_dtype),
            jax.ShapeDtypeStruct((MINI_BATCH,), jnp.float32),
            jax.ShapeDtypeStruct((MINI_BATCH, STATE_DIM), jnp.float32),
            jax.ShapeDtypeStruct((MINI_BATCH,), jnp.float32),
        ],
        scratch_types=[
            pltpu.VMEM((NCHUNK, CHUNK), jnp.int32),
            pltpu.VMEM((B_PER_W,), jnp.int32),
            pltpu.VMEM((NCHUNK, CHUNK), jnp.float32),
            pltpu.VMEM((NCHUNK, CHUNK), jnp.float32),
            pltpu.VMEM((NCHUNK, CHUNK), act_dtype),
            pltpu.SemaphoreType.DMA,
            pltpu.SemaphoreType.DMA,
            pltpu.SemaphoreType.DMA,
        ],
    )
    out_state, out_act, out_rew, out_next, out_msk = run(
        state, next_state, reward, masks, action, idx)
    return (out_state, out_act, out_rew, out_next, out_msk)


# speedup vs baseline: 1.0074x; 1.0074x over previous
import functools

import jax
import jax.numpy as jnp
from jax import lax
from jax.experimental import pallas as pl
from jax.experimental.pallas import tpu as pltpu
from jax.experimental.pallas import tpu_sc as plsc

MINI_BATCH = 16384
STATE_DIM = 64
NC = 2
NS = 16
NW = NC * NS
B_PER_W = MINI_BATCH // NW
NVEC = B_PER_W // 16


def _sample_body(st_flat, nx_flat, rew_hbm, msk_hbm, act_hbm, idx_hbm,
                 out_state, out_act, out_rew, out_next, out_msk,
                 idx_f, g_v, off_c, st_c, nx_c, rew_v, msk_v, act_v,
                 sem, sem2, sem3):
    wid = lax.axis_index("s") * NC + lax.axis_index("c")
    base = wid * B_PER_W

    pltpu.sync_copy(idx_hbm.at[pl.ds(base, B_PER_W)], idx_f)

    copies = [
        pltpu.async_copy(rew_hbm.at[idx_f], rew_v, sem3),
        pltpu.async_copy(msk_hbm.at[idx_f], msk_v, sem3),
        pltpu.async_copy(act_hbm.at[idx_f], act_v, sem3),
    ]

    @pl.loop(0, NVEC)
    def _g(k):
        v = idx_f[pl.ds(k * 16, 16)]
        g_v[pl.ds(k * 16, 16)] = lax.shift_left(v, 6)

    @pl.loop(0, STATE_DIM)
    def _off(j):
        for k in range(NVEC):
            off_c[j, pl.ds(k * 16, 16)] = g_v[pl.ds(k * 16, 16)] + j

    for j in range(STATE_DIM):
        copies.append(pltpu.async_copy(
            st_flat.at[off_c.at[j]], st_c.at[j], sem))
        copies.append(pltpu.async_copy(
            nx_flat.at[off_c.at[j]], nx_c.at[j], sem2))
    for cp in copies:
        cp.wait()

    pltpu.sync_copy(st_c, out_state.at[:, pl.ds(base, B_PER_W)])
    pltpu.sync_copy(nx_c, out_next.at[:, pl.ds(base, B_PER_W)])
    pltpu.sync_copy(rew_v, out_rew.at[pl.ds(base, B_PER_W)])
    pltpu.sync_copy(msk_v, out_msk.at[pl.ds(base, B_PER_W)])
    pltpu.sync_copy(act_v, out_act.at[pl.ds(base, B_PER_W)])


@jax.jit
def kernel(state, next_state, reward, masks, action, idx):
    idx = idx.astype(jnp.int32)
    act_dtype = action.dtype
    st_flat = jnp.reshape(state, (-1,))
    nx_flat = jnp.reshape(next_state, (-1,))
    mesh = plsc.VectorSubcoreMesh(core_axis_name="c", subcore_axis_name="s")
    run = pl.kernel(
        _sample_body,
        mesh=mesh,
        compiler_params=pltpu.CompilerParams(use_tc_tiling_on_sc=False),
        out_type=[
            jax.ShapeDtypeStruct((STATE_DIM, MINI_BATCH), jnp.float32),
            jax.ShapeDtypeStruct((MINI_BATCH,), act_dtype),
            jax.ShapeDtypeStruct((MINI_BATCH,), jnp.float32),
            jax.ShapeDtypeStruct((STATE_DIM, MINI_BATCH), jnp.float32),
            jax.ShapeDtypeStruct((MINI_BATCH,), jnp.float32),
        ],
        scratch_types=[
            pltpu.VMEM((B_PER_W,), jnp.int32),
            pltpu.VMEM((B_PER_W,), jnp.int32),
            pltpu.VMEM((STATE_DIM, B_PER_W), jnp.int32),
            pltpu.VMEM((STATE_DIM, B_PER_W), jnp.float32),
            pltpu.VMEM((STATE_DIM, B_PER_W), jnp.float32),
            pltpu.VMEM((B_PER_W,), jnp.float32),
            pltpu.VMEM((B_PER_W,), jnp.float32),
            pltpu.VMEM((B_PER_W,), act_dtype),
            pltpu.SemaphoreType.DMA,
            pltpu.SemaphoreType.DMA,
            pltpu.SemaphoreType.DMA,
        ],
    )
    out_state_t, out_act, out_rew, out_next_t, out_msk = run(
        st_flat, nx_flat, reward, masks, action, idx)
    return (out_state_t.T, out_act, out_rew, out_next_t.T, out_msk)
